# baseline (speedup 1.0000x reference)
import jax
import jax.numpy as jnp
from jax.experimental import pallas as pl
from jax.experimental.pallas import tpu as pltpu

T, H, E, I = 128, 768, 64, 768
HH = H // 2
EC = E // 2


def _moe_body(logits_ref, hidden_ref, wg_ref, wu_ref, w2a_ref, w2b_ref,
              out_ref, i1_ref, i2_ref, w1_ref, w2w_ref):
    c = pl.program_id(0)
    j = pl.program_id(1)
    e = c * EC + j

    @pl.when(j == 0)
    def _route():
        logits = logits_ref[...]
        m = jnp.max(logits, axis=1, keepdims=True)
        p = jnp.exp(logits - m)
        probs = p / jnp.sum(p, axis=1, keepdims=True)
        iota = jax.lax.broadcasted_iota(jnp.int32, (T, E), 1)
        m1 = jnp.max(probs, axis=1, keepdims=True)
        i1 = jnp.min(jnp.where(probs == m1, iota, E), axis=1, keepdims=True)
        oh1 = iota == i1
        pm = jnp.where(oh1, -jnp.inf, probs)
        m2 = jnp.max(pm, axis=1, keepdims=True)
        i2 = jnp.min(jnp.where(pm == m2, iota, E), axis=1, keepdims=True)
        denom = m1 + m2
        i1_ref[...] = i1
        i2_ref[...] = i2
        w1_ref[...] = m1 / denom
        w2w_ref[...] = m2 / denom
        out_ref[...] = jnp.zeros_like(out_ref)

    hs = hidden_ref[...]
    gate = jax.lax.dot_general(
        hs, wg_ref[0, 0], (((1,), (1,)), ((), ())),
        preferred_element_type=jnp.float32)
    up = jax.lax.dot_general(
        hs, wu_ref[0, 0], (((1,), (1,)), ((), ())),
        preferred_element_type=jnp.float32)
    act = gate * jax.lax.logistic(gate) * up
    eo_a = jax.lax.dot_general(
        act, w2a_ref[0, 0], (((1,), (1,)), ((), ())),
        preferred_element_type=jnp.float32)
    eo_b = jax.lax.dot_general(
        act, w2b_ref[0, 0], (((1,), (1,)), ((), ())),
        preferred_element_type=jnp.float32)
    col = (jnp.where(i1_ref[...] == e, w1_ref[...], 0.0)
           + jnp.where(i2_ref[...] == e, w2w_ref[...], 0.0))
    out_ref[0, :, :HH] += col * eo_a
    out_ref[0, :, HH:] += col * eo_b


def kernel(hidden_states, router_logits, w13, w2):
    w13v = w13.reshape(E, 2, I, H)
    w2v = w2.reshape(E, 2, HH, I)
    partial = pl.pallas_call(
        _moe_body,
        grid=(2, EC),
        in_specs=[
            pl.BlockSpec((T, E), lambda c, j: (0, 0)),
            pl.BlockSpec((T, H), lambda c, j: (0, 0)),
            pl.BlockSpec((1, 1, I, H), lambda c, j: (c * EC + j, 0, 0, 0)),
            pl.BlockSpec((1, 1, I, H), lambda c, j: (c * EC + j, 1, 0, 0)),
            pl.BlockSpec((1, 1, HH, I), lambda c, j: (c * EC + j, 0, 0, 0)),
            pl.BlockSpec((1, 1, HH, I), lambda c, j: (c * EC + j, 1, 0, 0)),
        ],
        out_specs=pl.BlockSpec((1, T, H), lambda c, j: (c, 0, 0)),
        out_shape=jax.ShapeDtypeStruct((2, T, H), jnp.float32),
        scratch_shapes=[
            pltpu.VMEM((T, 1), jnp.int32),
            pltpu.VMEM((T, 1), jnp.int32),
            pltpu.VMEM((T, 1), jnp.float32),
            pltpu.VMEM((T, 1), jnp.float32),
        ],
        compiler_params=pltpu.CompilerParams(
            dimension_semantics=("parallel", "arbitrary")),
    )(router_logits, hidden_states, w13v, w13v, w2v, w2v)
    return partial[0] + partial[1]


# speedup vs baseline: 1.0203x; 1.0203x over previous
import jax
import jax.numpy as jnp
from jax.experimental import pallas as pl
from jax.experimental.pallas import tpu as pltpu

T, H, E, I = 128, 768, 64, 768
HH = H // 2


def _moe_body(logits_ref, hidden_ref, w13_ref, w2_ref,
              out_ref, i1_ref, i2_ref, w1_ref, w2w_ref):
    e = pl.program_id(0)

    @pl.when(e == 0)
    def _route():
        logits = logits_ref[...]
        m = jnp.max(logits, axis=1, keepdims=True)
        p = jnp.exp(logits - m)
        probs = p / jnp.sum(p, axis=1, keepdims=True)
        iota = jax.lax.broadcasted_iota(jnp.int32, (T, E), 1)
        m1 = jnp.max(probs, axis=1, keepdims=True)
        i1 = jnp.min(jnp.where(probs == m1, iota, E), axis=1, keepdims=True)
        oh1 = iota == i1
        pm = jnp.where(oh1, -jnp.inf, probs)
        m2 = jnp.max(pm, axis=1, keepdims=True)
        i2 = jnp.min(jnp.where(pm == m2, iota, E), axis=1, keepdims=True)
        denom = m1 + m2
        i1_ref[...] = i1
        i2_ref[...] = i2
        w1_ref[...] = m1 / denom
        w2w_ref[...] = m2 / denom
        out_ref[...] = jnp.zeros_like(out_ref)

    hs = hidden_ref[...]
    gate = jax.lax.dot_general(
        hs, w13_ref[0, 0], (((1,), (1,)), ((), ())),
        preferred_element_type=jnp.float32)
    up = jax.lax.dot_general(
        hs, w13_ref[0, 1], (((1,), (1,)), ((), ())),
        preferred_element_type=jnp.float32)
    act = gate * jax.lax.logistic(gate) * up
    eo_a = jax.lax.dot_general(
        act, w2_ref[0, 0], (((1,), (1,)), ((), ())),
        preferred_element_type=jnp.float32)
    eo_b = jax.lax.dot_general(
        act, w2_ref[0, 1], (((1,), (1,)), ((), ())),
        preferred_element_type=jnp.float32)
    col = (jnp.where(i1_ref[...] == e, w1_ref[...], 0.0)
           + jnp.where(i2_ref[...] == e, w2w_ref[...], 0.0))
    out_ref[:, :HH] += col * eo_a
    out_ref[:, HH:] += col * eo_b


def kernel(hidden_states, router_logits, w13, w2):
    w13v = w13.reshape(E, 2, I, H)
    w2v = w2.reshape(E, 2, HH, I)
    return pl.pallas_call(
        _moe_body,
        grid=(E,),
        in_specs=[
            pl.BlockSpec((T, E), lambda e: (0, 0)),
            pl.BlockSpec((T, H), lambda e: (0, 0)),
            pl.BlockSpec((1, 2, I, H), lambda e: (e, 0, 0, 0)),
            pl.BlockSpec((1, 2, HH, I), lambda e: (e, 0, 0, 0)),
        ],
        out_specs=pl.BlockSpec((T, H), lambda e: (0, 0)),
        out_shape=jax.ShapeDtypeStruct((T, H), jnp.float32),
        scratch_shapes=[
            pltpu.VMEM((T, 1), jnp.int32),
            pltpu.VMEM((T, 1), jnp.int32),
            pltpu.VMEM((T, 1), jnp.float32),
            pltpu.VMEM((T, 1), jnp.float32),
        ],
    )(router_logits, hidden_states, w13v, w2v)


# speedup vs baseline: 1.0981x; 1.0763x over previous
import jax
import jax.numpy as jnp
from jax.experimental import pallas as pl
from jax.experimental.pallas import tpu as pltpu

T, H, E, I = 128, 768, 64, 768
HH = H // 2


def _moe_body(logits_ref, hidden_ref, w13_ref, w2_ref,
              out_ref, i1_ref, i2_ref, w1_ref, w2w_ref):
    g = pl.program_id(0)

    @pl.when(g == 0)
    def _route():
        logits = logits_ref[...]
        m = jnp.max(logits, axis=1, keepdims=True)
        p = jnp.exp(logits - m)
        probs = p / jnp.sum(p, axis=1, keepdims=True)
        iota = jax.lax.broadcasted_iota(jnp.int32, (T, E), 1)
        m1 = jnp.max(probs, axis=1, keepdims=True)
        i1 = jnp.min(jnp.where(probs == m1, iota, E), axis=1, keepdims=True)
        oh1 = iota == i1
        pm = jnp.where(oh1, -jnp.inf, probs)
        m2 = jnp.max(pm, axis=1, keepdims=True)
        i2 = jnp.min(jnp.where(pm == m2, iota, E), axis=1, keepdims=True)
        denom = m1 + m2
        i1_ref[...] = i1
        i2_ref[...] = i2
        w1_ref[...] = m1 / denom
        w2w_ref[...] = m2 / denom
        out_ref[...] = jnp.zeros_like(out_ref)

    hs = hidden_ref[...].astype(jnp.bfloat16)
    for k in range(2):
        e = g * 2 + k
        gate = jax.lax.dot_general(
            hs, w13_ref[k, 0].astype(jnp.bfloat16), (((1,), (1,)), ((), ())),
            preferred_element_type=jnp.float32)
        up = jax.lax.dot_general(
            hs, w13_ref[k, 1].astype(jnp.bfloat16), (((1,), (1,)), ((), ())),
            preferred_element_type=jnp.float32)
        act = (gate * jax.lax.logistic(gate) * up).astype(jnp.bfloat16)
        eo_a = jax.lax.dot_general(
            act, w2_ref[k, 0].astype(jnp.bfloat16), (((1,), (1,)), ((), ())),
            preferred_element_type=jnp.float32)
        eo_b = jax.lax.dot_general(
            act, w2_ref[k, 1].astype(jnp.bfloat16), (((1,), (1,)), ((), ())),
            preferred_element_type=jnp.float32)
        col = (jnp.where(i1_ref[...] == e, w1_ref[...], 0.0)
               + jnp.where(i2_ref[...] == e, w2w_ref[...], 0.0))
        out_ref[:, :HH] += col * eo_a
        out_ref[:, HH:] += col * eo_b


def kernel(hidden_states, router_logits, w13, w2):
    w13v = w13.reshape(E, 2, I, H)
    w2v = w2.reshape(E, 2, HH, I)
    return pl.pallas_call(
        _moe_body,
        grid=(E // 2,),
        in_specs=[
            pl.BlockSpec((T, E), lambda g: (0, 0)),
            pl.BlockSpec((T, H), lambda g: (0, 0)),
            pl.BlockSpec((2, 2, I, H), lambda g: (g, 0, 0, 0)),
            pl.BlockSpec((2, 2, HH, I), lambda g: (g, 0, 0, 0)),
        ],
        out_specs=pl.BlockSpec((T, H), lambda e: (0, 0)),
        out_shape=jax.ShapeDtypeStruct((T, H), jnp.float32),
        scratch_shapes=[
            pltpu.VMEM((T, 1), jnp.int32),
            pltpu.VMEM((T, 1), jnp.int32),
            pltpu.VMEM((T, 1), jnp.float32),
            pltpu.VMEM((T, 1), jnp.float32),
        ],
    )(router_logits, hidden_states, w13v, w2v)
